# Initial kernel scaffold; baseline (speedup 1.0000x reference)
import jax
import jax.numpy as jnp
from jax.experimental import pallas as pl


def kernel(x, edge_index, edge_weight, att, Wz, bz, Lz, lz_b, Wr, br, Lr, lr_b, Wh, bh, Lh, lh_b, fc_w, fc_b):
    raise NotImplementedError("write your pallas kernel here")


# speedup vs baseline: 108.2964x; 108.2964x over previous
import jax
import jax.numpy as jnp
from jax.experimental import pallas as pl
from jax.experimental.pallas import tpu as pltpu

N_NODES = 18
NP = 24
IN_F = 6
OUT_F = 64
PERIODS = 1536
N_EDGES = 306
E2 = N_EDGES + N_NODES
CHUNK = 128
NBLK = PERIODS // CHUNK

_HI = jax.lax.Precision.HIGHEST


def _prep_kernel(x_ref, ei_row_ref, ei_col_ref, ew_ref, att_ref, y_ref, probs_ref):
    loop_row = jax.lax.broadcasted_iota(jnp.int32, (1, N_NODES), 1)
    src_row = jnp.concatenate([ei_row_ref[0:1, :], loop_row], axis=1)
    dst_row = jnp.concatenate([ei_row_ref[1:2, :], loop_row], axis=1)
    loop_col = jax.lax.broadcasted_iota(jnp.int32, (N_NODES, 1), 0)
    src_col = jnp.concatenate([ei_col_ref[:, 0:1], loop_col], axis=0)
    w2 = jnp.concatenate(
        [ew_ref[0:1, :], jnp.ones((1, N_NODES), jnp.float32)], axis=1)

    node_iota = jax.lax.broadcasted_iota(jnp.int32, (N_NODES, E2), 0)
    dstM = (dst_row == node_iota).astype(jnp.float32)
    srcM = (src_row == node_iota).astype(jnp.float32)
    e_iota = jax.lax.broadcasted_iota(jnp.int32, (E2, N_NODES), 1)
    srcMT = (src_col == e_iota).astype(jnp.float32)

    deg = jnp.sum(dstM * w2, axis=1, keepdims=True)
    dinv = jnp.where(deg > 0, 1.0 / jnp.sqrt(deg), 0.0)
    dsrc = jnp.sum(srcM * dinv, axis=0, keepdims=True)
    ddst = jnp.sum(dstM * dinv, axis=0, keepdims=True)
    enorm = dsrc * w2 * ddst
    A = jnp.dot(dstM * enorm, srcMT,
                preferred_element_type=jnp.float32, precision=_HI)

    xx = x_ref[...]
    sq = [xx[:, f, :] * xx[:, f, :] for f in range(IN_F)]
    ssq = ((sq[0] + sq[4]) + sq[2]) + ((sq[1] + sq[5]) + sq[3])
    nrm = jnp.sqrt(ssq)[:, None, :]
    xn = xx / jnp.maximum(nrm, 1e-12)
    for f in range(IN_F):
        y_ref[:, f, :] = jnp.dot(A, xn[:, f, :],
                                 preferred_element_type=jnp.float32,
                                 precision=_HI)

    a = att_ref[...]
    e = jnp.exp(a - jnp.max(a))
    probs_ref[...] = e / jnp.sum(e)


def _main_kernel(probs_ref, ytp_ref, Wz_ref, Lz_ref, lzb_ref, Wr_ref, Lr_ref,
                 lrb_ref, Wh_ref, Lh_ref, lhb_ref, bz_ref, br_ref, bh_ref,
                 fcw_ref, fcb_ref, out_ref, P_ref, H_ref, Hacc_ref):
    i = pl.program_id(0)

    @pl.when(i == 0)
    def _init():
        H_ref[...] = jnp.zeros((NP, OUT_F), jnp.float32)
        Hacc_ref[...] = jnp.zeros((NP, OUT_F), jnp.float32)

    Wcat = jnp.concatenate([Wz_ref[...], Wr_ref[...], Wh_ref[...]], axis=1)
    bcat = jnp.concatenate([bz_ref[...], br_ref[...], bh_ref[...]], axis=1)
    P_ref[...] = jnp.dot(ytp_ref[...], Wcat,
                         preferred_element_type=jnp.float32,
                         precision=_HI) + bcat

    Lz = Lz_ref[...]
    Lr = Lr_ref[...]
    Lh = Lh_ref[...]
    lzb = lzb_ref[...]
    lrb = lrb_ref[...]
    lhb = lhb_ref[...]
    base_t = i * CHUNK

    def body(t, carry):
        H, Hacc = carry
        Pt = P_ref[pl.ds(t * NP, NP), :]
        Z = jax.nn.sigmoid(
            jnp.dot(jnp.concatenate([Pt[:, :OUT_F], H], axis=1), Lz,
                    preferred_element_type=jnp.float32) + lzb)
        R = jax.nn.sigmoid(
            jnp.dot(jnp.concatenate([Pt[:, OUT_F:2 * OUT_F], H], axis=1), Lr,
                    preferred_element_type=jnp.float32) + lrb)
        Ht = jnp.tanh(
            jnp.dot(jnp.concatenate([Pt[:, 2 * OUT_F:], H * R], axis=1), Lh,
                    preferred_element_type=jnp.float32) + lhb)
        Hn = Z * H + (1.0 - Z) * Ht
        p = probs_ref[0, base_t + t]
        return Hn, Hacc + p * Hn

    H, Hacc = jax.lax.fori_loop(
        0, CHUNK, body, (H_ref[...], Hacc_ref[...]))
    H_ref[...] = H
    Hacc_ref[...] = Hacc

    @pl.when(i == NBLK - 1)
    def _fin():
        hrelu = jnp.maximum(Hacc[:N_NODES, :], 0.0)
        out_ref[0, 0] = jnp.sum(hrelu * fcw_ref[...]) + fcb_ref[0, 0]


def kernel(x, edge_index, edge_weight, att, Wz, bz, Lz, lz_b, Wr, br, Lr,
           lr_b, Wh, bh, Lh, lh_b, fc_w, fc_b):
    y, probs = pl.pallas_call(
        _prep_kernel,
        out_shape=[
            jax.ShapeDtypeStruct((N_NODES, IN_F, PERIODS), jnp.float32),
            jax.ShapeDtypeStruct((1, PERIODS), jnp.float32),
        ],
    )(x, edge_index, edge_index.T, edge_weight.reshape(1, N_EDGES),
      att.reshape(1, PERIODS))

    yt = jnp.transpose(y, (2, 0, 1))
    ytp = jnp.pad(yt, ((0, 0), (0, NP - N_NODES), (0, 0)))
    ytp = ytp.reshape(PERIODS * NP, IN_F)

    full = lambda shape: pl.BlockSpec(shape, lambda i: tuple(0 for _ in shape))
    out = pl.pallas_call(
        _main_kernel,
        grid=(NBLK,),
        in_specs=[
            pl.BlockSpec(memory_space=pltpu.SMEM),
            pl.BlockSpec((CHUNK * NP, IN_F), lambda i: (i, 0)),
            full((IN_F, OUT_F)), full((2 * OUT_F, OUT_F)), full((1, OUT_F)),
            full((IN_F, OUT_F)), full((2 * OUT_F, OUT_F)), full((1, OUT_F)),
            full((IN_F, OUT_F)), full((2 * OUT_F, OUT_F)), full((1, OUT_F)),
            full((1, OUT_F)), full((1, OUT_F)), full((1, OUT_F)),
            full((N_NODES, OUT_F)), full((1, 1)),
        ],
        out_specs=pl.BlockSpec((1, 1), lambda i: (0, 0),
                               memory_space=pltpu.SMEM),
        out_shape=jax.ShapeDtypeStruct((1, 1), jnp.float32),
        scratch_shapes=[
            pltpu.VMEM((CHUNK * NP, 3 * OUT_F), jnp.float32),
            pltpu.VMEM((NP, OUT_F), jnp.float32),
            pltpu.VMEM((NP, OUT_F), jnp.float32),
        ],
    )(probs, ytp, Wz, Lz, lz_b.reshape(1, OUT_F), Wr, Lr,
      lr_b.reshape(1, OUT_F), Wh, Lh, lh_b.reshape(1, OUT_F),
      bz.reshape(1, OUT_F), br.reshape(1, OUT_F), bh.reshape(1, OUT_F),
      fc_w.reshape(N_NODES, OUT_F), fc_b.reshape(1, 1))
    return out.reshape(1)
